# baseline (speedup 1.0000x reference)
import functools

import jax
import jax.numpy as jnp
from jax import lax
from jax.experimental import pallas as pl
from jax.experimental.pallas import tpu as pltpu
from jax.experimental.pallas import tpu_sc as plsc

_NC = 2
_NS = 16
_NW = _NC * _NS
_CB = 16
_TBLK = 2048


def _tc_transpose(table_t):
    d, v = table_t.shape
    grid = (v + _TBLK - 1) // _TBLK

    def body(i_ref, o_ref):
        o_ref[...] = lax.dot_general(
            i_ref[...],
            jnp.eye(d, dtype=jnp.float32),
            (((0,), (0,)), ((), ())),
            preferred_element_type=jnp.float32,
        )

    return pl.pallas_call(
        body,
        grid=(grid,),
        in_specs=[pl.BlockSpec((d, _TBLK), lambda i: (0, i))],
        out_specs=pl.BlockSpec((_TBLK, d), lambda i: (i, 0)),
        out_shape=jax.ShapeDtypeStruct((v, d), jnp.float32),
    )(table_t)


def _sc_gather(q, t1):
    b, l = q.shape
    d = t1.shape[1]
    rows_per_w = b // _NW
    nchunk = rows_per_w // _CB
    mesh = plsc.VectorSubcoreMesh(core_axis_name="c", subcore_axis_name="s")

    @functools.partial(
        pl.kernel,
        out_type=jax.ShapeDtypeStruct((b, d), jnp.float32),
        mesh=mesh,
        scratch_types=[
            pltpu.VMEM((2, _CB, l), jnp.int32),
            pltpu.VMEM((2, _CB * l, d), jnp.float32),
            pltpu.VMEM((2, _CB, d), jnp.float32),
            pltpu.SemaphoreType.DMA,
            pltpu.SemaphoreType.DMA,
        ],
        compiler_params=pltpu.CompilerParams(use_tc_tiling_on_sc=False),
    )
    def k(q_hbm, t_hbm, out_hbm, idx_v, rows_v, out_v, sem0, sem1):
        wid = lax.axis_index("s") * _NC + lax.axis_index("c")
        base_row = wid * rows_per_w
        sems = [sem0, sem1]

        beta = [
            (lax.iota(jnp.int32, 16).astype(jnp.float32) + (16.0 * kk - 31.0))
            * (1.0 / 800.0)
            for kk in range(d // 16)
        ]

        def fire(ci, buf):
            row0 = base_row + ci * _CB
            pltpu.sync_copy(q_hbm.at[pl.ds(row0, _CB)], idx_v.at[buf])
            for j in range(_CB):
                pltpu.async_copy(
                    t_hbm.at[idx_v.at[buf].at[j]],
                    rows_v.at[buf].at[pl.ds(j * l, l)],
                    sems[buf],
                )

        def drain(buf):
            pltpu.make_async_copy(
                t_hbm.at[pl.ds(0, _CB * l)], rows_v.at[buf], sems[buf]
            ).wait()

        def compute(ci, buf):
            rows = rows_v.at[buf]
            row0 = base_row + ci * _CB

            def row_body(r, carry2):
                acc0 = [None] * (d // 16)
                acc1 = [None] * (d // 16)
                for li in range(l):
                    alpha = float(li - 24)
                    for kk in range(d // 16):
                        v = rows[r * l + li, pl.ds(16 * kk, 16)]
                        if li == 0:
                            acc0[kk] = v
                            acc1[kk] = alpha * v
                        else:
                            acc0[kk] = acc0[kk] + v
                            if alpha == 1.0:
                                acc1[kk] = acc1[kk] + v
                            elif alpha != 0.0:
                                acc1[kk] = acc1[kk] + alpha * v
                for kk in range(d // 16):
                    out_v[buf, r, pl.ds(16 * kk, 16)] = (
                        acc0[kk] + beta[kk] * acc1[kk]
                    )
                return carry2

            lax.fori_loop(0, _CB, row_body, 0)
            pltpu.sync_copy(out_v.at[buf], out_hbm.at[pl.ds(row0, _CB)])

        fire(0, 0)

        def pair_body(p, carry):
            ci0 = p * 2
            for bb in range(2):
                ci = ci0 + bb
                nxt = ci + 1

                @pl.when(nxt < nchunk)
                def _():
                    fire(nxt, 1 - bb)

                drain(bb)
                compute(ci, bb)
            return carry

        lax.fori_loop(0, nchunk // 2, pair_body, 0)

    return k(q, t1)


def kernel(questions, table):
    q = questions.astype(jnp.int32)
    t1 = _tc_transpose(table.T)
    return _sc_gather(q, t1)


# speedup vs baseline: 1.3925x; 1.3925x over previous
import functools

import jax
import jax.numpy as jnp
from jax import lax
from jax.experimental import pallas as pl
from jax.experimental.pallas import tpu as pltpu
from jax.experimental.pallas import tpu_sc as plsc

_NC = 2
_NS = 16
_NW = _NC * _NS
_CB = 16


def _sc_call(questions, table):
    b, l = questions.shape
    d = table.shape[1]
    nk = d // 16
    rows_per_w = b // _NW
    nchunk = rows_per_w // _CB
    mesh = plsc.VectorSubcoreMesh(core_axis_name="c", subcore_axis_name="s")

    @functools.partial(
        pl.kernel,
        out_type=jax.ShapeDtypeStruct((b, d), jnp.float32),
        mesh=mesh,
        scratch_types=[
            pltpu.VMEM((2, _CB, l), jnp.int32),
            pltpu.VMEM((2, _CB * l, d), jnp.float32),
            pltpu.VMEM((2, _CB, d), jnp.float32),
            pltpu.SemaphoreType.DMA,
            pltpu.SemaphoreType.DMA,
        ],
        compiler_params=pltpu.CompilerParams(use_tc_tiling_on_sc=False),
    )
    def k(q_hbm, t_hbm, out_hbm, idx_v, rows_v, out_v, sem0, sem1):
        wid = lax.axis_index("s") * _NC + lax.axis_index("c")
        base_row = wid * rows_per_w
        sems = [sem0, sem1]

        beta = [
            (lax.iota(jnp.int32, 16).astype(jnp.float32) + (16.0 * kk - 31.0))
            * (1.0 / 800.0)
            for kk in range(nk)
        ]

        def fire(ci, buf):
            row0 = base_row + ci * _CB
            pltpu.sync_copy(q_hbm.at[pl.ds(row0, _CB)], idx_v.at[buf])
            for j in range(_CB):
                pltpu.async_copy(
                    t_hbm.at[idx_v.at[buf].at[j]],
                    rows_v.at[buf].at[pl.ds(j * l, l)],
                    sems[buf],
                )

        def drain(buf):
            pltpu.make_async_copy(
                t_hbm.at[pl.ds(0, _CB * l)], rows_v.at[buf], sems[buf]
            ).wait()

        def compute(ci, buf):
            rows = rows_v.at[buf]
            row0 = base_row + ci * _CB

            def row_body(r, carry):
                base = r * l
                acc0 = [None] * nk
                acc1 = [None] * nk
                for li in range(l):
                    alpha = float(li - 24)
                    for kk in range(nk):
                        v = rows[base + li, pl.ds(16 * kk, 16)]
                        if li == 0:
                            acc0[kk] = v
                            acc1[kk] = alpha * v
                        else:
                            acc0[kk] = acc0[kk] + v
                            if alpha == 1.0:
                                acc1[kk] = acc1[kk] + v
                            elif alpha != 0.0:
                                acc1[kk] = acc1[kk] + alpha * v
                for kk in range(nk):
                    out_v[buf, r, pl.ds(16 * kk, 16)] = (
                        acc0[kk] + beta[kk] * acc1[kk]
                    )
                return carry

            lax.fori_loop(0, _CB, row_body, 0)
            pltpu.sync_copy(out_v.at[buf], out_hbm.at[pl.ds(row0, _CB)])

        fire(0, 0)

        def pair_body(p, carry):
            ci0 = p * 2
            for bb in range(2):
                ci = ci0 + bb
                nxt = ci + 1

                @pl.when(nxt < nchunk)
                def _():
                    fire(nxt, 1 - bb)

                drain(bb)
                compute(ci, bb)
            return carry

        lax.fori_loop(0, nchunk // 2, pair_body, 0)

    return k(questions, table)


def kernel(questions, table):
    q = questions.astype(jnp.int32)
    return _sc_call(q, table)
